# baseline (speedup 1.0000x reference)
import functools

import jax
import jax.numpy as jnp
from jax import lax
from jax.experimental import pallas as pl
from jax.experimental.pallas import tpu as pltpu
from jax.experimental.pallas import tpu_sc as plsc

_B = 4096
_L = 200
_D = 128
_N_TOTAL = _B * _L
_NC = 2
_NS = 16
_NW = _NC * _NS
_W = _N_TOTAL // _NW
_G = 128
_GRP = 2 * _G
_NGRP = _W // _GRP
_NOUT = _NGRP // 2


@jax.jit
def _gather(idx_flat, table):
    mesh = plsc.VectorSubcoreMesh(core_axis_name="c", subcore_axis_name="s")

    @functools.partial(
        pl.kernel,
        mesh=mesh,
        out_type=jax.ShapeDtypeStruct((_N_TOTAL, _D), jnp.float32),
        scratch_types=[
            pltpu.VMEM((_W,), jnp.int32),
            pltpu.VMEM((2 * _GRP, _D), jnp.float32),
            pltpu.SemaphoreType.DMA((4,)),
            pltpu.SemaphoreType.DMA((2,)),
        ],
    )
    def k(idx_hbm, tab_hbm, out_hbm, idx_v, rows_v, gsem, osem):
        wid = lax.axis_index("s") * _NC + lax.axis_index("c")
        base = wid * _W

        pltpu.sync_copy(idx_hbm.at[pl.ds(base, _W)], idx_v)

        def outer(m, _):
            for p in range(2):
                g = 2 * m + p
                @pl.when(m > 0)
                def _drain():
                    pltpu.make_async_copy(
                        rows_v.at[pl.ds(p * _GRP, _GRP)],
                        out_hbm.at[pl.ds(base + (g - 2) * _GRP, _GRP)],
                        osem.at[p]).wait()
                for b in range(2):
                    pltpu.async_copy(
                        tab_hbm.at[idx_v.at[pl.ds(g * _GRP + b * _G, _G)]],
                        rows_v.at[pl.ds(p * _GRP + b * _G, _G)],
                        gsem.at[2 * p + b])
            for p in range(2):
                g = 2 * m + p
                for b in range(2):
                    pltpu.make_async_copy(
                        tab_hbm.at[idx_v.at[pl.ds(g * _GRP + b * _G, _G)]],
                        rows_v.at[pl.ds(p * _GRP + b * _G, _G)],
                        gsem.at[2 * p + b]).wait()
                pltpu.async_copy(
                    rows_v.at[pl.ds(p * _GRP, _GRP)],
                    out_hbm.at[pl.ds(base + g * _GRP, _GRP)], osem.at[p])
            return 0

        lax.fori_loop(0, _NOUT, outer, 0)

        for p in range(2):
            g = 2 * (_NOUT - 1) + p
            pltpu.make_async_copy(
                rows_v.at[pl.ds(p * _GRP, _GRP)],
                out_hbm.at[pl.ds(base + g * _GRP, _GRP)],
                osem.at[p]).wait()

    return k(idx_flat, table)


def kernel(word_idx, emb0, emb1):
    del emb1
    out = _gather(word_idx.reshape(_N_TOTAL), emb0)
    return out.reshape(_B, _L, _D)
